# baseline (speedup 1.0000x reference)
import jax
import jax.numpy as jnp
from jax.experimental import pallas as pl

N = 4096
TM = 512


def _body(prev_ref, nblk_ref, ntT_ref, a0_ref, a1_ref, wphi_ref, wth_ref,
          out_ref):
    i = pl.program_id(0)
    w0 = wth_ref[0, 0]
    w1 = wth_ref[1, 0]
    w2 = wth_ref[2, 0]
    c0 = w0 * w0
    c1 = w1 * w1
    c2 = w2 * w2

    x0 = ntT_ref[0:1, :]
    x1 = ntT_ref[1:2, :]
    x2 = ntT_ref[2:3, :]
    g0 = x0 * (-2.0 * c0)
    g1 = x1 * (-2.0 * c1)
    g2 = x2 * (-2.0 * c2)
    sq = x0 * x0 * c0 + x1 * x1 * c1 + x2 * x2 * c2

    y0 = nblk_ref[:, 0:1]
    y1 = nblk_ref[:, 1:2]
    y2 = nblk_ref[:, 2:3]

    z = ((sq + y0 * g0) + y1 * g1) + y2 * g2

    neg = jnp.float32(-jnp.inf)
    h = N // 2
    m0 = jnp.max(jnp.where(a0_ref[:, :] != 0, z[:, :h], neg),
                 axis=1, keepdims=True)
    m1 = jnp.max(jnp.where(a1_ref[:, :] != 0, z[:, h:], neg),
                 axis=1, keepdims=True)
    m = jnp.maximum(m0, m1)
    mrow = m.T
    xi0 = ntT_ref[0:1, pl.ds(i * TM, TM)]
    xi1 = ntT_ref[1:2, pl.ds(i * TM, TM)]
    xi2 = ntT_ref[2:3, pl.ds(i * TM, TM)]
    sqi = xi0 * xi0 * c0 + xi1 * xi1 * c1 + xi2 * xi2 * c2
    d2 = sqi + mrow
    maxd = jnp.where(mrow == neg, neg, jnp.sqrt(jnp.maximum(d2, 0.0)))

    half_wmean = 0.5 * jnp.mean(wphi_ref[0, :])
    out_ref[0:1, :] = 0.5 * prev_ref[0:1, :] + maxd * half_wmean


@jax.jit
def _run(prev, nodes, adjacency, wphi, wth):
    prev = prev.reshape(1, N)
    wphi = wphi.reshape(1, -1)
    ntT = nodes.T
    grid = (N // TM,)
    out = pl.pallas_call(
        _body,
        grid=grid,
        in_specs=[
            pl.BlockSpec((1, TM), lambda i: (0, i)),
            pl.BlockSpec((TM, 3), lambda i: (i, 0)),
            pl.BlockSpec((3, N), lambda i: (0, 0)),
            pl.BlockSpec((TM, N // 2), lambda i: (i, 0)),
            pl.BlockSpec((TM, N // 2), lambda i: (i, 1)),
            pl.BlockSpec((1, wphi.shape[1]), lambda i: (0, 0)),
            pl.BlockSpec((3, 1), lambda i: (0, 0)),
        ],
        out_specs=pl.BlockSpec((1, TM), lambda i: (0, i)),
        out_shape=jax.ShapeDtypeStruct((1, N), jnp.float32),
    )(prev, nodes, ntT, adjacency, adjacency, wphi, wth)
    return out.reshape(N)


def kernel(previous_inclusion_score, nodes, adjacency_matrix, W_phi, W_theta):
    return _run(previous_inclusion_score, nodes, adjacency_matrix, W_phi,
                W_theta)
